# baseline (speedup 1.0000x reference)
import jax
import jax.numpy as jnp
from jax.experimental import pallas as pl
from jax.experimental.pallas import tpu as pltpu


def _logits_softmax_body(k_ref, r_ref, q_ref, w0_ref, w1_ref, a0_ref, a1_ref):
    t = pl.program_id(1)
    tc = k_ref.shape[0]

    K = k_ref[...]
    r = r_ref[...]
    q0 = q_ref[0]
    q1 = q_ref[1]

    a0_ref[pl.ds(t * tc, tc), :] = jnp.sum(K * q0[None, :, :], axis=2) * r
    a1_ref[pl.ds(t * tc, tc), :] = jnp.sum(K * q1[None, :, :], axis=2) * r

    @pl.when(t == pl.num_programs(1) - 1)
    def _():
        a0 = a0_ref[...]
        e0 = jnp.exp(a0 - jnp.max(a0, axis=0, keepdims=True))
        w0_ref[...] = e0 / jnp.sum(e0, axis=0, keepdims=True)
        a1 = a1_ref[...]
        e1 = jnp.exp(a1 - jnp.max(a1, axis=0, keepdims=True))
        w1_ref[...] = e1 / jnp.sum(e1, axis=0, keepdims=True)


def _wsum_linear_body(w0_ref, w1_ref, v_ref, W_ref, b_ref, out_ref, acc_ref):
    t = pl.program_id(1)

    @pl.when(t == 0)
    def _():
        acc_ref[...] = jnp.zeros_like(acc_ref)

    vc = v_ref[...]
    w0 = w0_ref[...]
    w1 = w1_ref[...]
    D = vc.shape[2]
    acc_ref[:, :D] += jnp.sum(w0[:, :, None] * vc, axis=0)
    acc_ref[:, D:] += jnp.sum(w1[:, :, None] * vc, axis=0)

    @pl.when(t == pl.num_programs(1) - 1)
    def _():
        out_ref[...] = jax.lax.dot_general(
            acc_ref[...], W_ref[...],
            (((1,), (1,)), ((), ())),
            preferred_element_type=jnp.float32,
            precision=jax.lax.Precision.HIGHEST,
        ) + b_ref[...]


def kernel(keys, vals, rpe, query, W, b):
    T, B, E = keys.shape
    D = vals.shape[2]
    H = query.shape[1]

    BB = 128
    TC = 40
    NB = B // BB
    NT = T // TC

    rpe2 = rpe.reshape(T, B)
    qT = jnp.transpose(query, (1, 0, 2))
    b2 = b.reshape(1, D)

    w0, w1 = pl.pallas_call(
        _logits_softmax_body,
        grid=(NB, NT),
        in_specs=[
            pl.BlockSpec((TC, BB, E), lambda i, t: (t, i, 0)),
            pl.BlockSpec((TC, BB), lambda i, t: (t, i)),
            pl.BlockSpec((H, BB, E), lambda i, t: (0, i, 0)),
        ],
        out_specs=[
            pl.BlockSpec((T, BB), lambda i, t: (0, i)),
            pl.BlockSpec((T, BB), lambda i, t: (0, i)),
        ],
        out_shape=[
            jax.ShapeDtypeStruct((T, B), jnp.float32),
            jax.ShapeDtypeStruct((T, B), jnp.float32),
        ],
        scratch_shapes=[
            pltpu.VMEM((T, BB), jnp.float32),
            pltpu.VMEM((T, BB), jnp.float32),
        ],
        compiler_params=pltpu.CompilerParams(
            dimension_semantics=("parallel", "arbitrary"),
        ),
    )(keys, rpe2, qT)

    out = pl.pallas_call(
        _wsum_linear_body,
        grid=(NB, NT),
        in_specs=[
            pl.BlockSpec((TC, BB), lambda i, t: (t, i)),
            pl.BlockSpec((TC, BB), lambda i, t: (t, i)),
            pl.BlockSpec((TC, BB, D), lambda i, t: (t, i, 0)),
            pl.BlockSpec((D, H * D), lambda i, t: (0, 0)),
            pl.BlockSpec((1, D), lambda i, t: (0, 0)),
        ],
        out_specs=pl.BlockSpec((BB, D), lambda i, t: (i, 0)),
        out_shape=jax.ShapeDtypeStruct((B, D), jnp.float32),
        scratch_shapes=[pltpu.VMEM((BB, H * D), jnp.float32)],
        compiler_params=pltpu.CompilerParams(
            dimension_semantics=("parallel", "arbitrary"),
        ),
    )(w0, w1, vals, W, b2)

    return out


# speedup vs baseline: 1.1585x; 1.1585x over previous
import functools
import jax
import jax.numpy as jnp
from jax import lax
from jax.experimental import pallas as pl
from jax.experimental.pallas import tpu as pltpu
from jax.experimental.pallas import tpu_sc as plsc

_NC, _NS, _L = 2, 16, 16
_NW = _NC * _NS


def _logits_softmax_body(k_ref, r_ref, q_ref, w0_ref, w1_ref, a0_ref, a1_ref):
    t = pl.program_id(1)
    tc = k_ref.shape[0]

    Kt = jnp.swapaxes(k_ref[...], 1, 2)
    r = r_ref[...]
    q0 = q_ref[0]
    q1 = q_ref[1]

    a0_ref[pl.ds(t * tc, tc), :] = jnp.sum(Kt * q0[None, :, :], axis=1) * r
    a1_ref[pl.ds(t * tc, tc), :] = jnp.sum(Kt * q1[None, :, :], axis=1) * r

    @pl.when(t == pl.num_programs(1) - 1)
    def _():
        a0 = a0_ref[...]
        e0 = jnp.exp(a0 - jnp.max(a0, axis=0, keepdims=True))
        w0_ref[...] = jnp.swapaxes(e0 / jnp.sum(e0, axis=0, keepdims=True), 0, 1)
        a1 = a1_ref[...]
        e1 = jnp.exp(a1 - jnp.max(a1, axis=0, keepdims=True))
        w1_ref[...] = jnp.swapaxes(e1 / jnp.sum(e1, axis=0, keepdims=True), 0, 1)


def _splat(v, j):
    idx = jnp.full((16, 1), j, jnp.int32)
    return lax.gather(
        v, idx,
        lax.GatherDimensionNumbers(
            offset_dims=(), collapsed_slice_dims=(0,), start_index_map=(0,)),
        (1,),
        mode=lax.GatherScatterMode.PROMISE_IN_BOUNDS)


def _make_sc_wsum(T, B, D):
    TP = 208
    NB = B // _NW
    DC = D // _L
    HALF = TP // 2

    mesh = plsc.VectorSubcoreMesh(core_axis_name="c", subcore_axis_name="s")

    @functools.partial(
        pl.kernel,
        mesh=mesh,
        out_type=[
            jax.ShapeDtypeStruct((B, D), jnp.float32),
            jax.ShapeDtypeStruct((B, D), jnp.float32),
        ],
        scratch_types=[
            pltpu.VMEM((TP, D), jnp.float32),
            pltpu.VMEM((TP, D), jnp.float32),
            pltpu.VMEM((TP,), jnp.float32),
            pltpu.VMEM((TP,), jnp.float32),
            pltpu.VMEM((TP,), jnp.float32),
            pltpu.VMEM((TP,), jnp.float32),
            pltpu.VMEM((TP,), jnp.int32),
            pltpu.VMEM((TP,), jnp.int32),
            pltpu.VMEM((NB, D), jnp.float32),
            pltpu.VMEM((NB, D), jnp.float32),
            pltpu.SemaphoreType.DMA,
            pltpu.SemaphoreType.DMA,
        ],
    )
    def sc_wsum(vals_hbm, w0_hbm, w1_hbm, res0_hbm, res1_hbm,
                rows_a, rows_b, w0a, w0b, w1a, w1b, idx_a, idx_b,
                r0_v, r1_v, sem0, sem1):
        wid = lax.axis_index("c") * _NS + lax.axis_index("s")
        b0 = wid * NB
        ibase = lax.iota(jnp.int32, 16) * B
        zeros16 = jnp.zeros((16,), jnp.float32)
        bufs = ((rows_a, w0a, w1a, idx_a, sem0),
                (rows_b, w0b, w1b, idx_b, sem1))

        def prep_and_fire(nb, p):
            rows_v, w0_v, w1_v, idx_v, sem = bufs[p]
            bg = b0 + nb
            woff = pl.multiple_of(bg * T, 8)
            top = (T - 1) * B + bg
            for c in range(TP // 16):
                idx_v[pl.ds(c * 16, 16)] = jnp.minimum(
                    ibase + (c * 16 * B + bg), top)
            w0_v[pl.ds(TP - 16, 16)] = zeros16
            w1_v[pl.ds(TP - 16, 16)] = zeros16
            pltpu.async_copy(w0_hbm.at[pl.ds(woff, T)], w0_v.at[pl.ds(0, T)], sem)
            pltpu.async_copy(w1_hbm.at[pl.ds(woff, T)], w1_v.at[pl.ds(0, T)], sem)
            pltpu.async_copy(vals_hbm.at[idx_v.at[pl.ds(0, HALF)]],
                             rows_v.at[pl.ds(0, HALF)], sem)
            pltpu.async_copy(vals_hbm.at[idx_v.at[pl.ds(HALF, HALF)]],
                             rows_v.at[pl.ds(HALF, HALF)], sem)

        def wait_all(nb, p):
            rows_v, w0_v, w1_v, idx_v, sem = bufs[p]
            bg = b0 + nb
            woff = pl.multiple_of(bg * T, 8)
            pltpu.make_async_copy(w0_hbm.at[pl.ds(woff, T)], w0_v.at[pl.ds(0, T)], sem).wait()
            pltpu.make_async_copy(w1_hbm.at[pl.ds(woff, T)], w1_v.at[pl.ds(0, T)], sem).wait()
            pltpu.make_async_copy(vals_hbm.at[idx_v.at[pl.ds(0, HALF)]],
                                  rows_v.at[pl.ds(0, HALF)], sem).wait()
            pltpu.make_async_copy(vals_hbm.at[idx_v.at[pl.ds(HALF, HALF)]],
                                  rows_v.at[pl.ds(HALF, HALF)], sem).wait()

        def compute(nb, p):
            rows_v, w0_v, w1_v, idx_v, sem = bufs[p]

            def chunk_body(c, carry):
                acc0, acc1 = carry
                wv0 = w0_v[pl.ds(c * 16, 16)]
                wv1 = w1_v[pl.ds(c * 16, 16)]
                tbase = c * 16
                for j in range(16):
                    s0 = _splat(wv0, j)
                    s1 = _splat(wv1, j)
                    t = tbase + j
                    acc0 = [a + s0 * rows_v[t, pl.ds(k * _L, _L)]
                            for k, a in enumerate(acc0)]
                    acc1 = [a + s1 * rows_v[t, pl.ds(k * _L, _L)]
                            for k, a in enumerate(acc1)]
                return acc0, acc1

            zero = [jnp.zeros((_L,), jnp.float32)] * DC
            acc0, acc1 = lax.fori_loop(0, TP // 16, chunk_body,
                                       (list(zero), list(zero)))
            for k in range(DC):
                r0_v[nb, pl.ds(k * _L, _L)] = acc0[k]
                r1_v[nb, pl.ds(k * _L, _L)] = acc1[k]

        prep_and_fire(0, 0)

        def pair_body(i, carry):
            nb = i * 2

            wait_all(nb, 0)

            @pl.when(nb + 1 < NB)
            def _fire1():
                prep_and_fire(nb + 1, 1)
            compute(nb, 0)

            @pl.when(nb + 1 < NB)
            def _second():
                wait_all(nb + 1, 1)

                @pl.when(nb + 2 < NB)
                def _fire0():
                    prep_and_fire(nb + 2, 0)
                compute(nb + 1, 1)
            return carry

        lax.fori_loop(0, (NB + 1) // 2, pair_body, 0)

        pltpu.sync_copy(r0_v, res0_hbm.at[pl.ds(b0, NB)])
        pltpu.sync_copy(r1_v, res1_hbm.at[pl.ds(b0, NB)])

    return sc_wsum


def _linear_body(r0_ref, r1_ref, W_ref, b_ref, out_ref):
    D = out_ref.shape[1]
    Wf = W_ref[...]
    out_ref[...] = (
        jax.lax.dot_general(r0_ref[...], Wf[:, :D], (((1,), (1,)), ((), ())),
                            preferred_element_type=jnp.float32,
                            precision=jax.lax.Precision.HIGHEST)
        + jax.lax.dot_general(r1_ref[...], Wf[:, D:], (((1,), (1,)), ((), ())),
                              preferred_element_type=jnp.float32,
                              precision=jax.lax.Precision.HIGHEST)
        + b_ref[...]
    )


def kernel(keys, vals, rpe, query, W, b):
    T, B, E = keys.shape
    D = vals.shape[2]
    H = query.shape[1]

    BB = 128
    TC = 40
    NB = B // BB
    NT = T // TC

    rpe2 = rpe.reshape(T, B)
    qT = jnp.transpose(query, (1, 2, 0))
    b2 = b.reshape(1, D)

    w0, w1 = pl.pallas_call(
        _logits_softmax_body,
        grid=(NB, NT),
        in_specs=[
            pl.BlockSpec((TC, BB, E), lambda i, t: (t, i, 0)),
            pl.BlockSpec((TC, BB), lambda i, t: (t, i)),
            pl.BlockSpec((H, E, BB), lambda i, t: (0, 0, i)),
        ],
        out_specs=[
            pl.BlockSpec((BB, T), lambda i, t: (i, 0)),
            pl.BlockSpec((BB, T), lambda i, t: (i, 0)),
        ],
        out_shape=[
            jax.ShapeDtypeStruct((B, T), jnp.float32),
            jax.ShapeDtypeStruct((B, T), jnp.float32),
        ],
        scratch_shapes=[
            pltpu.VMEM((T, BB), jnp.float32),
            pltpu.VMEM((T, BB), jnp.float32),
        ],
        compiler_params=pltpu.CompilerParams(
            dimension_semantics=("parallel", "arbitrary"),
        ),
    )(keys, rpe2, qT)

    res0, res1 = _make_sc_wsum(T, B, D)(
        vals.reshape(T * B, D), w0.reshape(B * T), w1.reshape(B * T))

    BL = 256
    out = pl.pallas_call(
        _linear_body,
        grid=(B // BL,),
        in_specs=[
            pl.BlockSpec((BL, D), lambda i: (i, 0)),
            pl.BlockSpec((BL, D), lambda i: (i, 0)),
            pl.BlockSpec((D, H * D), lambda i: (0, 0)),
            pl.BlockSpec((1, D), lambda i: (0, 0)),
        ],
        out_specs=pl.BlockSpec((BL, D), lambda i: (i, 0)),
        out_shape=jax.ShapeDtypeStruct((B, D), jnp.float32),
        compiler_params=pltpu.CompilerParams(
            dimension_semantics=("parallel",),
        ),
    )(res0, res1, W, b2)

    return out


# speedup vs baseline: 1.1604x; 1.0017x over previous
import functools
import jax
import jax.numpy as jnp
from jax import lax
from jax.experimental import pallas as pl
from jax.experimental.pallas import tpu as pltpu
from jax.experimental.pallas import tpu_sc as plsc

_NC, _NS, _L = 2, 16, 16
_NW = _NC * _NS


def _logits_softmax_body(k_ref, r_ref, q_ref, w0_ref, w1_ref, a0_ref, a1_ref):
    t = pl.program_id(1)
    tc = k_ref.shape[0]

    Kt = jnp.swapaxes(k_ref[...], 1, 2)
    r = r_ref[...]
    q0 = q_ref[0]
    q1 = q_ref[1]

    a0_ref[pl.ds(t * tc, tc), :] = jnp.sum(Kt * q0[None, :, :], axis=1) * r
    a1_ref[pl.ds(t * tc, tc), :] = jnp.sum(Kt * q1[None, :, :], axis=1) * r

    @pl.when(t == pl.num_programs(1) - 1)
    def _():
        a0 = a0_ref[...]
        e0 = jnp.exp(a0 - jnp.max(a0, axis=0, keepdims=True))
        w0_ref[...] = jnp.swapaxes(e0 / jnp.sum(e0, axis=0, keepdims=True), 0, 1)
        a1 = a1_ref[...]
        e1 = jnp.exp(a1 - jnp.max(a1, axis=0, keepdims=True))
        w1_ref[...] = jnp.swapaxes(e1 / jnp.sum(e1, axis=0, keepdims=True), 0, 1)


def _splat(v, j):
    idx = jnp.full((16, 1), j, jnp.int32)
    return lax.gather(
        v, idx,
        lax.GatherDimensionNumbers(
            offset_dims=(), collapsed_slice_dims=(0,), start_index_map=(0,)),
        (1,),
        mode=lax.GatherScatterMode.PROMISE_IN_BOUNDS)


def _make_sc_wsum(T, B, D):
    TP = 208
    NB = B // _NW
    DC = D // _L
    HALF = TP // 2

    mesh = plsc.VectorSubcoreMesh(core_axis_name="c", subcore_axis_name="s")

    @functools.partial(
        pl.kernel,
        mesh=mesh,
        out_type=[
            jax.ShapeDtypeStruct((B, D), jnp.float32),
            jax.ShapeDtypeStruct((B, D), jnp.float32),
        ],
        scratch_types=[
            pltpu.VMEM((TP, D), jnp.float32),
            pltpu.VMEM((TP, D), jnp.float32),
            pltpu.VMEM((TP,), jnp.float32),
            pltpu.VMEM((TP,), jnp.float32),
            pltpu.VMEM((TP,), jnp.float32),
            pltpu.VMEM((TP,), jnp.float32),
            pltpu.VMEM((TP,), jnp.int32),
            pltpu.VMEM((TP,), jnp.int32),
            pltpu.VMEM((NB, D), jnp.float32),
            pltpu.VMEM((NB, D), jnp.float32),
            pltpu.SemaphoreType.DMA,
            pltpu.SemaphoreType.DMA,
        ],
    )
    def sc_wsum(vals3_hbm, w0_hbm, w1_hbm, res0_hbm, res1_hbm,
                rows_a, rows_b, w0a, w0b, w1a, w1b, idx_a, idx_b,
                r0_v, r1_v, sem0, sem1):
        vals_hbm = vals3_hbm.reshape(T * B, D)
        wid = lax.axis_index("c") * _NS + lax.axis_index("s")
        b0 = wid * NB
        ibase = lax.iota(jnp.int32, 16) * B
        zeros16 = jnp.zeros((16,), jnp.float32)
        bufs = ((rows_a, w0a, w1a, idx_a, sem0),
                (rows_b, w0b, w1b, idx_b, sem1))

        def prep_and_fire(nb, p):
            rows_v, w0_v, w1_v, idx_v, sem = bufs[p]
            bg = b0 + nb
            woff = pl.multiple_of(bg * T, 8)
            top = (T - 1) * B + bg
            for c in range(TP // 16):
                idx_v[pl.ds(c * 16, 16)] = jnp.minimum(
                    ibase + (c * 16 * B + bg), top)
            w0_v[pl.ds(TP - 16, 16)] = zeros16
            w1_v[pl.ds(TP - 16, 16)] = zeros16
            pltpu.async_copy(w0_hbm.at[pl.ds(woff, T)], w0_v.at[pl.ds(0, T)], sem)
            pltpu.async_copy(w1_hbm.at[pl.ds(woff, T)], w1_v.at[pl.ds(0, T)], sem)
            pltpu.async_copy(vals_hbm.at[idx_v.at[pl.ds(0, HALF)]],
                             rows_v.at[pl.ds(0, HALF)], sem)
            pltpu.async_copy(vals_hbm.at[idx_v.at[pl.ds(HALF, HALF)]],
                             rows_v.at[pl.ds(HALF, HALF)], sem)

        def wait_all(nb, p):
            rows_v, w0_v, w1_v, idx_v, sem = bufs[p]
            bg = b0 + nb
            woff = pl.multiple_of(bg * T, 8)
            pltpu.make_async_copy(w0_hbm.at[pl.ds(woff, T)], w0_v.at[pl.ds(0, T)], sem).wait()
            pltpu.make_async_copy(w1_hbm.at[pl.ds(woff, T)], w1_v.at[pl.ds(0, T)], sem).wait()
            pltpu.make_async_copy(vals_hbm.at[idx_v.at[pl.ds(0, HALF)]],
                                  rows_v.at[pl.ds(0, HALF)], sem).wait()
            pltpu.make_async_copy(vals_hbm.at[idx_v.at[pl.ds(HALF, HALF)]],
                                  rows_v.at[pl.ds(HALF, HALF)], sem).wait()

        def compute(nb, p):
            rows_v, w0_v, w1_v, idx_v, sem = bufs[p]

            def chunk_body(c, carry):
                acc0, acc1 = carry
                wv0 = w0_v[pl.ds(c * 16, 16)]
                wv1 = w1_v[pl.ds(c * 16, 16)]
                tbase = c * 16
                for j in range(16):
                    s0 = _splat(wv0, j)
                    s1 = _splat(wv1, j)
                    t = tbase + j
                    acc0 = [a + s0 * rows_v[t, pl.ds(k * _L, _L)]
                            for k, a in enumerate(acc0)]
                    acc1 = [a + s1 * rows_v[t, pl.ds(k * _L, _L)]
                            for k, a in enumerate(acc1)]
                return acc0, acc1

            zero = [jnp.zeros((_L,), jnp.float32)] * DC
            acc0, acc1 = lax.fori_loop(0, TP // 16, chunk_body,
                                       (list(zero), list(zero)))
            for k in range(DC):
                r0_v[nb, pl.ds(k * _L, _L)] = acc0[k]
                r1_v[nb, pl.ds(k * _L, _L)] = acc1[k]

        prep_and_fire(0, 0)

        def pair_body(i, carry):
            nb = i * 2

            wait_all(nb, 0)

            @pl.when(nb + 1 < NB)
            def _fire1():
                prep_and_fire(nb + 1, 1)
            compute(nb, 0)

            @pl.when(nb + 1 < NB)
            def _second():
                wait_all(nb + 1, 1)

                @pl.when(nb + 2 < NB)
                def _fire0():
                    prep_and_fire(nb + 2, 0)
                compute(nb + 1, 1)
            return carry

        lax.fori_loop(0, (NB + 1) // 2, pair_body, 0)

        pltpu.sync_copy(r0_v, res0_hbm.at[pl.ds(b0, NB)])
        pltpu.sync_copy(r1_v, res1_hbm.at[pl.ds(b0, NB)])

    return sc_wsum


def _linear_body(r0_ref, r1_ref, W_ref, b_ref, out_ref):
    D = out_ref.shape[1]
    Wf = W_ref[...]
    out_ref[...] = (
        jax.lax.dot_general(r0_ref[...], Wf[:, :D], (((1,), (1,)), ((), ())),
                            preferred_element_type=jnp.float32,
                            precision=jax.lax.Precision.HIGHEST)
        + jax.lax.dot_general(r1_ref[...], Wf[:, D:], (((1,), (1,)), ((), ())),
                              preferred_element_type=jnp.float32,
                              precision=jax.lax.Precision.HIGHEST)
        + b_ref[...]
    )


def kernel(keys, vals, rpe, query, W, b):
    T, B, E = keys.shape
    D = vals.shape[2]
    H = query.shape[1]

    BB = 128
    TC = 40
    NB = B // BB
    NT = T // TC

    rpe2 = rpe.reshape(T, B)
    qT = jnp.transpose(query, (1, 2, 0))
    b2 = b.reshape(1, D)

    w0, w1 = pl.pallas_call(
        _logits_softmax_body,
        grid=(NB, NT),
        in_specs=[
            pl.BlockSpec((TC, BB, E), lambda i, t: (t, i, 0)),
            pl.BlockSpec((TC, BB), lambda i, t: (t, i)),
            pl.BlockSpec((H, E, BB), lambda i, t: (0, 0, i)),
        ],
        out_specs=[
            pl.BlockSpec((BB, T), lambda i, t: (i, 0)),
            pl.BlockSpec((BB, T), lambda i, t: (i, 0)),
        ],
        out_shape=[
            jax.ShapeDtypeStruct((B, T), jnp.float32),
            jax.ShapeDtypeStruct((B, T), jnp.float32),
        ],
        scratch_shapes=[
            pltpu.VMEM((T, BB), jnp.float32),
            pltpu.VMEM((T, BB), jnp.float32),
        ],
        compiler_params=pltpu.CompilerParams(
            dimension_semantics=("parallel", "arbitrary"),
        ),
    )(keys, rpe2, qT)

    res0, res1 = _make_sc_wsum(T, B, D)(
        vals, w0.reshape(B * T), w1.reshape(B * T))

    BL = 256
    out = pl.pallas_call(
        _linear_body,
        grid=(B // BL,),
        in_specs=[
            pl.BlockSpec((BL, D), lambda i: (i, 0)),
            pl.BlockSpec((BL, D), lambda i: (i, 0)),
            pl.BlockSpec((D, H * D), lambda i: (0, 0)),
            pl.BlockSpec((1, D), lambda i: (0, 0)),
        ],
        out_specs=pl.BlockSpec((BL, D), lambda i: (i, 0)),
        out_shape=jax.ShapeDtypeStruct((B, D), jnp.float32),
        compiler_params=pltpu.CompilerParams(
            dimension_semantics=("parallel",),
        ),
    )(res0, res1, W, b2)

    return out


# speedup vs baseline: 1.1934x; 1.0284x over previous
import functools
import jax
import jax.numpy as jnp
from jax import lax
from jax.experimental import pallas as pl
from jax.experimental.pallas import tpu as pltpu
from jax.experimental.pallas import tpu_sc as plsc

_NC, _NS, _L = 2, 16, 16
_NW = _NC * _NS


def _logits_softmax_body(k_ref, r_ref, q_ref, w0_ref, w1_ref, a0_ref, a1_ref):
    t = pl.program_id(1)
    tc = k_ref.shape[0]

    Kt = jnp.swapaxes(k_ref[...], 1, 2)
    r = r_ref[...]
    q0 = q_ref[0]
    q1 = q_ref[1]

    a0_ref[pl.ds(t * tc, tc), :] = jnp.sum(Kt * q0[None, :, :], axis=1) * r
    a1_ref[pl.ds(t * tc, tc), :] = jnp.sum(Kt * q1[None, :, :], axis=1) * r

    @pl.when(t == pl.num_programs(1) - 1)
    def _():
        a0 = a0_ref[...]
        e0 = jnp.exp(a0 - jnp.max(a0, axis=0, keepdims=True))
        w0_ref[...] = jnp.swapaxes(e0 / jnp.sum(e0, axis=0, keepdims=True), 0, 1)
        a1 = a1_ref[...]
        e1 = jnp.exp(a1 - jnp.max(a1, axis=0, keepdims=True))
        w1_ref[...] = jnp.swapaxes(e1 / jnp.sum(e1, axis=0, keepdims=True), 0, 1)


def _splat(v, j):
    idx = jnp.full((16, 1), j, jnp.int32)
    return lax.gather(
        v, idx,
        lax.GatherDimensionNumbers(
            offset_dims=(), collapsed_slice_dims=(0,), start_index_map=(0,)),
        (1,),
        mode=lax.GatherScatterMode.PROMISE_IN_BOUNDS)


def _make_sc_wsum(T, B, D):
    TP = 208
    NB = B // _NW
    DC = D // _L
    HALF = TP // 2

    mesh = plsc.VectorSubcoreMesh(core_axis_name="c", subcore_axis_name="s")

    @functools.partial(
        pl.kernel,
        mesh=mesh,
        out_type=[
            jax.ShapeDtypeStruct((B, D), jnp.float32),
            jax.ShapeDtypeStruct((B, D), jnp.float32),
        ],
        scratch_types=[
            pltpu.VMEM((200, D), jnp.float32),
            pltpu.VMEM((200, D), jnp.float32),
            pltpu.VMEM((6416,), jnp.float32),
            pltpu.VMEM((6416,), jnp.float32),
            pltpu.VMEM((TP,), jnp.int32),
            pltpu.VMEM((TP,), jnp.int32),
            pltpu.VMEM((NB // 2, D), jnp.float32),
            pltpu.VMEM((NB // 2, D), jnp.float32),
            pltpu.SemaphoreType.DMA,
            pltpu.SemaphoreType.DMA,
            pltpu.SemaphoreType.DMA,
        ],
    )
    def sc_wsum(vals3_hbm, w0_hbm, w1_hbm, res0_hbm, res1_hbm,
                rows_a, rows_b, w0_v, w1_v, idx_a, idx_b,
                r0_v, r1_v, sem0, sem1, semw):
        vals_hbm = vals3_hbm.reshape(T * B, D)
        wid = lax.axis_index("c") * _NS + lax.axis_index("s")
        b0 = wid * NB
        ibase = lax.iota(jnp.int32, 16) * B
        bufs = ((rows_a, idx_a, sem0), (rows_b, idx_b, sem1))
        NH = NB // 2
        CH = T // 16
        REM = T % 16
        WLEN = NB * T

        woff0 = pl.multiple_of(b0 * T, 8)
        pltpu.async_copy(w0_hbm.at[pl.ds(woff0, WLEN)],
                         w0_v.at[pl.ds(0, WLEN)], semw)
        pltpu.async_copy(w1_hbm.at[pl.ds(woff0, WLEN)],
                         w1_v.at[pl.ds(0, WLEN)], semw)
        pltpu.make_async_copy(w0_hbm.at[pl.ds(woff0, WLEN)],
                              w0_v.at[pl.ds(0, WLEN)], semw).wait()
        pltpu.make_async_copy(w1_hbm.at[pl.ds(woff0, WLEN)],
                              w1_v.at[pl.ds(0, WLEN)], semw).wait()

        def prep_and_fire(nb, p):
            rows_v, idx_v, sem = bufs[p]
            bg = b0 + nb
            top = (T - 1) * B + bg
            for c in range(TP // 16):
                idx_v[pl.ds(c * 16, 16)] = jnp.minimum(
                    ibase + (c * 16 * B + bg), top)
            pltpu.async_copy(vals_hbm.at[idx_v.at[pl.ds(0, HALF)]],
                             rows_v.at[pl.ds(0, HALF)], sem)
            pltpu.async_copy(vals_hbm.at[idx_v.at[pl.ds(HALF, T - HALF)]],
                             rows_v.at[pl.ds(HALF, T - HALF)], sem)

        def wait_rows(p):
            rows_v, idx_v, sem = bufs[p]
            pltpu.make_async_copy(vals_hbm.at[idx_v.at[pl.ds(0, HALF)]],
                                  rows_v.at[pl.ds(0, HALF)], sem).wait()
            pltpu.make_async_copy(vals_hbm.at[idx_v.at[pl.ds(HALF, T - HALF)]],
                                  rows_v.at[pl.ds(HALF, T - HALF)], sem).wait()

        def compute(nb, slot, p):
            rows_v, idx_v, sem = bufs[p]
            wbase = nb * T

            def chunk_body(c, carry):
                acc0, acc1 = carry
                wv0 = w0_v[pl.ds(wbase + c * 16, 16)]
                wv1 = w1_v[pl.ds(wbase + c * 16, 16)]
                tbase = c * 16
                for j in range(16):
                    s0 = _splat(wv0, j)
                    s1 = _splat(wv1, j)
                    t = tbase + j
                    acc0 = [a + s0 * rows_v[t, pl.ds(k * _L, _L)]
                            for k, a in enumerate(acc0)]
                    acc1 = [a + s1 * rows_v[t, pl.ds(k * _L, _L)]
                            for k, a in enumerate(acc1)]
                return acc0, acc1

            zero = [jnp.zeros((_L,), jnp.float32)] * DC
            acc0, acc1 = lax.fori_loop(0, CH, chunk_body,
                                       (list(zero), list(zero)))
            wv0 = w0_v[pl.ds(wbase + CH * 16, 16)]
            wv1 = w1_v[pl.ds(wbase + CH * 16, 16)]
            for j in range(REM):
                s0 = _splat(wv0, j)
                s1 = _splat(wv1, j)
                t = CH * 16 + j
                acc0 = [a + s0 * rows_v[t, pl.ds(k * _L, _L)]
                        for k, a in enumerate(acc0)]
                acc1 = [a + s1 * rows_v[t, pl.ds(k * _L, _L)]
                        for k, a in enumerate(acc1)]
            for k in range(DC):
                r0_v[slot, pl.ds(k * _L, _L)] = acc0[k]
                r1_v[slot, pl.ds(k * _L, _L)] = acc1[k]

        for half in range(2):
            base = half * NH
            prep_and_fire(base, 0)

            def pair_body(i, carry, base=base):
                nb = base + i * 2
                wait_rows(0)
                prep_and_fire(nb + 1, 1)
                compute(nb, nb - base, 0)
                wait_rows(1)

                @pl.when(i < NH // 2 - 1)
                def _fire0():
                    prep_and_fire(nb + 2, 0)
                compute(nb + 1, nb + 1 - base, 1)
                return carry

            lax.fori_loop(0, NH // 2, pair_body, 0)
            pltpu.sync_copy(r0_v, res0_hbm.at[pl.ds(b0 + base, NH)])
            pltpu.sync_copy(r1_v, res1_hbm.at[pl.ds(b0 + base, NH)])

    return sc_wsum


def _linear_body(r0_ref, r1_ref, W_ref, b_ref, out_ref):
    D = out_ref.shape[1]
    Wf = W_ref[...]
    out_ref[...] = (
        jax.lax.dot_general(r0_ref[...], Wf[:, :D], (((1,), (1,)), ((), ())),
                            preferred_element_type=jnp.float32,
                            precision=jax.lax.Precision.HIGHEST)
        + jax.lax.dot_general(r1_ref[...], Wf[:, D:], (((1,), (1,)), ((), ())),
                              preferred_element_type=jnp.float32,
                              precision=jax.lax.Precision.HIGHEST)
        + b_ref[...]
    )


def kernel(keys, vals, rpe, query, W, b):
    T, B, E = keys.shape
    D = vals.shape[2]
    H = query.shape[1]

    BB = 128
    TC = 40
    NB = B // BB
    NT = T // TC

    rpe2 = rpe.reshape(T, B)
    qT = jnp.transpose(query, (1, 2, 0))
    b2 = b.reshape(1, D)

    w0, w1 = pl.pallas_call(
        _logits_softmax_body,
        grid=(NB, NT),
        in_specs=[
            pl.BlockSpec((TC, BB, E), lambda i, t: (t, i, 0)),
            pl.BlockSpec((TC, BB), lambda i, t: (t, i)),
            pl.BlockSpec((H, E, BB), lambda i, t: (0, 0, i)),
        ],
        out_specs=[
            pl.BlockSpec((BB, T), lambda i, t: (i, 0)),
            pl.BlockSpec((BB, T), lambda i, t: (i, 0)),
        ],
        out_shape=[
            jax.ShapeDtypeStruct((B, T), jnp.float32),
            jax.ShapeDtypeStruct((B, T), jnp.float32),
        ],
        scratch_shapes=[
            pltpu.VMEM((T, BB), jnp.float32),
            pltpu.VMEM((T, BB), jnp.float32),
        ],
        compiler_params=pltpu.CompilerParams(
            dimension_semantics=("parallel", "arbitrary"),
        ),
    )(keys, rpe2, qT)

    res0, res1 = _make_sc_wsum(T, B, D)(
        vals, w0.reshape(B * T), w1.reshape(B * T))

    BL = 256
    out = pl.pallas_call(
        _linear_body,
        grid=(B // BL,),
        in_specs=[
            pl.BlockSpec((BL, D), lambda i: (i, 0)),
            pl.BlockSpec((BL, D), lambda i: (i, 0)),
            pl.BlockSpec((D, H * D), lambda i: (0, 0)),
            pl.BlockSpec((1, D), lambda i: (0, 0)),
        ],
        out_specs=pl.BlockSpec((BL, D), lambda i: (i, 0)),
        out_shape=jax.ShapeDtypeStruct((B, D), jnp.float32),
        compiler_params=pltpu.CompilerParams(
            dimension_semantics=("parallel",),
        ),
    )(res0, res1, W, b2)

    return out
